# baseline (speedup 1.0000x reference)
import jax
import jax.numpy as jnp
from jax import lax
from jax.experimental import pallas as pl
from jax.experimental.pallas import tpu as pltpu
from jax.experimental.pallas import tpu_sc as plsc

NROWS = 100000
TBLK = 2048
B = 1024
K = 50
KP = 56
D = 1032
L = 16
NC = 2
NS = 16
NW = NC * NS
SPB = B // NW

NCH = 3
CW = D // NCH
NWIN = CW // L
TAIL_OFF = CW - L
WINDOWS = [c * L for c in range(NWIN)] + ([TAIL_OFF] if CW % L else [])


def _tp_body(in_ref, out_ref):
    out_ref[...] = in_ref[...].T


def _tp_chunk(wT, ci):
    return pl.pallas_call(
        _tp_body,
        grid=(pl.cdiv(NROWS, TBLK),),
        in_specs=[pl.BlockSpec((CW, TBLK), lambda i, ci=ci: (ci, i))],
        out_specs=pl.BlockSpec((TBLK, CW), lambda i: (i, 0)),
        out_shape=jax.ShapeDtypeStruct((NROWS, CW), jnp.float32),
    )(wT)


def _sc_chunk_body(fi0, fv0, fi1, fv1, w, bias, out0, out1,
                   idxv, valv, rows0, rows1, biasv, obuf, sem0, sem1):
    wid = lax.axis_index("s") * NC + lax.axis_index("c")
    base = wid * SPB

    pltpu.sync_copy(bias, biasv)

    def run_side(fi, fv, out):
        pltpu.sync_copy(fi.at[pl.ds(base, SPB)], idxv)
        pltpu.sync_copy(fv.at[pl.ds(base * KP, SPB * KP)], valv)

        def gather(i, buf, sem):
            return pltpu.make_async_copy(w.at[idxv.at[i]], buf, sem)

        def compute(rbuf, i):
            j = lax.rem(i, 8)
            vbase = jnp.full((L,), i * KP, jnp.int32)

            def k_body(k, accs):
                v = plsc.load_gather(valv, [vbase + k])
                return tuple(a + v * rbuf[k, pl.ds(o, L)]
                             for a, o in zip(accs, WINDOWS))

            init = tuple(biasv[pl.ds(o, L)] for o in WINDOWS)
            accs = lax.fori_loop(0, K, k_body, init)
            for a, o in zip(accs, WINDOWS):
                obuf[j, pl.ds(o, L)] = a

            @pl.when(j == 7)
            def _():
                gb = pl.multiple_of(base + i - 7, 8)
                pltpu.sync_copy(obuf, out.at[pl.ds(gb, 8)])

        gather(0, rows0, sem0).start()

        def body(j, _):
            i = 2 * j
            gather(i + 1, rows1, sem1).start()
            gather(i, rows0, sem0).wait()
            compute(rows0, i)

            @pl.when(j < SPB // 2 - 1)
            def _():
                gather(i + 2, rows0, sem0).start()

            gather(i + 1, rows1, sem1).wait()
            compute(rows1, i + 1)
            return 0

        lax.fori_loop(0, SPB // 2, body, 0)

    run_side(fi0, fv0, out0)
    run_side(fi1, fv1, out1)


@jax.jit
def _transform(fi0, fv0, fi1, fv1, wT, merged_bias):
    f32 = jnp.float32
    mesh = plsc.VectorSubcoreMesh(core_axis_name="c", subcore_axis_name="s")
    parts0, parts1 = [], []
    for ci in range(NCH):
        w_c = _tp_chunk(wT, ci)
        bias_c = lax.dynamic_slice(merged_bias, (ci * CW,), (CW,))
        o0, o1 = pl.kernel(
            _sc_chunk_body,
            out_type=(jax.ShapeDtypeStruct((B, CW), f32),
                      jax.ShapeDtypeStruct((B, CW), f32)),
            mesh=mesh,
            compiler_params=pltpu.CompilerParams(
                needs_layout_passes=False, use_tc_tiling_on_sc=False),
            scratch_types=[
                pltpu.VMEM((SPB, KP), jnp.int32),
                pltpu.VMEM((SPB * KP,), f32),
                pltpu.VMEM((KP, CW), f32),
                pltpu.VMEM((KP, CW), f32),
                pltpu.VMEM((CW,), f32),
                pltpu.VMEM((8, CW), f32),
                pltpu.SemaphoreType.DMA,
                pltpu.SemaphoreType.DMA,
            ],
        )(fi0, fv0, fi1, fv1, w_c, bias_c)
        parts0.append(o0)
        parts1.append(o1)
    return (jnp.concatenate(parts0, axis=1), jnp.concatenate(parts1, axis=1))


def kernel(feature_indices_0, feature_values_0, feature_indices_1,
           feature_values_1, weight, bias_ft, bias_psqt):
    pad = ((0, 0), (0, KP - K))
    fi0 = jnp.pad(feature_indices_0, pad)
    fi1 = jnp.pad(feature_indices_1, pad)
    fv0 = jnp.pad(feature_values_0, pad).reshape(-1)
    fv1 = jnp.pad(feature_values_1, pad).reshape(-1)
    merged_bias = jnp.concatenate([bias_ft, bias_psqt], axis=0)
    return _transform(fi0, fv0, fi1, fv1, weight.T, merged_bias)


# speedup vs baseline: 1.0017x; 1.0017x over previous
import jax
import jax.numpy as jnp
from jax import lax
from jax.experimental import pallas as pl
from jax.experimental.pallas import tpu as pltpu
from jax.experimental.pallas import tpu_sc as plsc

NROWS = 100000
TBLK = 2048
B = 1024
K = 50
KP = 56
D = 1032
L = 16
NC = 2
NS = 16
NW = NC * NS
SPB = B // NW

NCH = 3
CW = D // NCH
NWIN = CW // L
TAIL_OFF = CW - L
WINDOWS = [c * L for c in range(NWIN)] + ([TAIL_OFF] if CW % L else [])


def _tp_body(in_ref, out_ref):
    out_ref[...] = in_ref[...].T


def _tp_chunk(wT, ci):
    return pl.pallas_call(
        _tp_body,
        grid=(pl.cdiv(NROWS, TBLK),),
        in_specs=[pl.BlockSpec((CW, TBLK), lambda i, ci=ci: (ci, i))],
        out_specs=pl.BlockSpec((TBLK, CW), lambda i: (i, 0)),
        out_shape=jax.ShapeDtypeStruct((NROWS, CW), jnp.float32),
        compiler_params=pltpu.CompilerParams(skip_device_barrier=True),
    )(wT)


def _sc_chunk_body(fi0, fv0, fi1, fv1, w, bias, out0, out1,
                   idxv, valv, rows0, rows1, biasv, obuf, sem0, sem1):
    wid = lax.axis_index("s") * NC + lax.axis_index("c")
    base = wid * SPB

    pltpu.sync_copy(bias, biasv)

    def run_side(fi, fv, out):
        pltpu.sync_copy(fi.at[pl.ds(base, SPB)], idxv)
        pltpu.sync_copy(fv.at[pl.ds(base * KP, SPB * KP)], valv)

        def gather(i, buf, sem):
            return pltpu.make_async_copy(w.at[idxv.at[i]], buf, sem)

        def compute(rbuf, i):
            j = lax.rem(i, 8)
            vbase = jnp.full((L,), i * KP, jnp.int32)

            def k_body(k, accs):
                v = plsc.load_gather(valv, [vbase + k])
                return tuple(a + v * rbuf[k, pl.ds(o, L)]
                             for a, o in zip(accs, WINDOWS))

            init = tuple(biasv[pl.ds(o, L)] for o in WINDOWS)
            accs = lax.fori_loop(0, K, k_body, init)
            for a, o in zip(accs, WINDOWS):
                obuf[j, pl.ds(o, L)] = a

            @pl.when(j == 7)
            def _():
                gb = pl.multiple_of(base + i - 7, 8)
                pltpu.sync_copy(obuf, out.at[pl.ds(gb, 8)])

        gather(0, rows0, sem0).start()

        def body(j, _):
            i = 2 * j
            gather(i + 1, rows1, sem1).start()
            gather(i, rows0, sem0).wait()
            compute(rows0, i)

            @pl.when(j < SPB // 2 - 1)
            def _():
                gather(i + 2, rows0, sem0).start()

            gather(i + 1, rows1, sem1).wait()
            compute(rows1, i + 1)
            return 0

        lax.fori_loop(0, SPB // 2, body, 0)

    run_side(fi0, fv0, out0)
    run_side(fi1, fv1, out1)


@jax.jit
def _transform(fi0, fv0, fi1, fv1, wT, merged_bias):
    f32 = jnp.float32
    mesh = plsc.VectorSubcoreMesh(core_axis_name="c", subcore_axis_name="s")
    parts0, parts1 = [], []
    for ci in range(NCH):
        w_c = _tp_chunk(wT, ci)
        bias_c = lax.dynamic_slice(merged_bias, (ci * CW,), (CW,))
        o0, o1 = pl.kernel(
            _sc_chunk_body,
            out_type=(jax.ShapeDtypeStruct((B, CW), f32),
                      jax.ShapeDtypeStruct((B, CW), f32)),
            mesh=mesh,
            compiler_params=pltpu.CompilerParams(
                needs_layout_passes=False, use_tc_tiling_on_sc=False,
                skip_device_barrier=True),
            scratch_types=[
                pltpu.VMEM((SPB, KP), jnp.int32),
                pltpu.VMEM((SPB * KP,), f32),
                pltpu.VMEM((KP, CW), f32),
                pltpu.VMEM((KP, CW), f32),
                pltpu.VMEM((CW,), f32),
                pltpu.VMEM((8, CW), f32),
                pltpu.SemaphoreType.DMA,
                pltpu.SemaphoreType.DMA,
            ],
        )(fi0, fv0, fi1, fv1, w_c, bias_c)
        parts0.append(o0)
        parts1.append(o1)
    return (jnp.concatenate(parts0, axis=1), jnp.concatenate(parts1, axis=1))


def kernel(feature_indices_0, feature_values_0, feature_indices_1,
           feature_values_1, weight, bias_ft, bias_psqt):
    pad = ((0, 0), (0, KP - K))
    fi0 = jnp.pad(feature_indices_0, pad)
    fi1 = jnp.pad(feature_indices_1, pad)
    fv0 = jnp.pad(feature_values_0, pad).reshape(-1)
    fv1 = jnp.pad(feature_values_1, pad).reshape(-1)
    merged_bias = jnp.concatenate([bias_ft, bias_psqt], axis=0)
    return _transform(fi0, fv0, fi1, fv1, weight.T, merged_bias)


# speedup vs baseline: 1.2974x; 1.2952x over previous
import jax
import jax.numpy as jnp
from jax import lax
from jax.experimental import pallas as pl
from jax.experimental.pallas import tpu as pltpu
from jax.experimental.pallas import tpu_sc as plsc

NROWS = 100000
TBLK = 2048
B = 1024
K = 50
KP = 56
D = 1032
L = 16
W = 32
NC = 2
NS = 16
NW = NC * NS
SPB = B // NW
NFULL = D // W
TAIL_OFF = D - W

WINDOWS = [c * W for c in range(NFULL)] + [TAIL_OFF]
HALVES = (WINDOWS[:16], WINDOWS[16:])


def _sc_body(fi0, fv0, fi1, fv1, w, bias, out0, out1,
             idxv, valv, rows0, rows1, biasv, obuf, sem0, sem1):
    wid = lax.axis_index("s") * NC + lax.axis_index("c")
    base = wid * SPB

    pltpu.sync_copy(bias, biasv)
    lanes = lax.iota(jnp.int32, L)

    def run_side(fi, fv, out):
        pltpu.sync_copy(fi.at[pl.ds(base, SPB)], idxv)
        pltpu.sync_copy(fv.at[pl.ds(base * KP, SPB * KP)], valv)

        def gather(i, buf, sem):
            return pltpu.make_async_copy(w.at[idxv.at[i]], buf, sem)

        def compute(rbuf, i):
            j = lax.rem(i, 8)
            jv = jnp.full((L,), j, jnp.int32)
            vbase = jnp.full((L,), i * KP, jnp.int32)
            for half in HALVES:
                def k_body(k, accs):
                    v = plsc.load_gather(valv, [vbase + k])
                    new = []
                    for (ae, ao), o in zip(accs, half):
                        be, bo = plsc.unpack(rbuf[k, pl.ds(o, W)],
                                             format=plsc.PackFormat.INTERLEAVED)
                        new.append((ae + v * be, ao + v * bo))
                    return tuple(new)

                zero = jnp.zeros((L,), jnp.float32)
                init = tuple((zero, zero) for _ in half)
                accs = lax.fori_loop(0, K, k_body, init)
                for (ae, ao), o in zip(accs, half):
                    cols = jnp.full((L,), o, jnp.int32) + 2 * lanes
                    fe = ae + plsc.load_gather(biasv, [cols])
                    fo = ao + plsc.load_gather(biasv, [cols + 1])
                    plsc.store_scatter(obuf, [jv, cols], fe)
                    plsc.store_scatter(obuf, [jv, cols + 1], fo)

            @pl.when(j == 7)
            def _():
                gb = pl.multiple_of(base + i - 7, 8)
                pltpu.sync_copy(obuf, out.at[pl.ds(gb, 8)])

        gather(0, rows0, sem0).start()

        def body(j, _):
            i = 2 * j
            gather(i + 1, rows1, sem1).start()
            gather(i, rows0, sem0).wait()
            compute(rows0, i)

            @pl.when(j < SPB // 2 - 1)
            def _():
                gather(i + 2, rows0, sem0).start()

            gather(i + 1, rows1, sem1).wait()
            compute(rows1, i + 1)
            return 0

        lax.fori_loop(0, SPB // 2, body, 0)

    run_side(fi0, fv0, out0)
    run_side(fi1, fv1, out1)


@jax.jit
def _transform(fi0, fv0, fi1, fv1, w, merged_bias):
    f32 = jnp.float32
    mesh = plsc.VectorSubcoreMesh(core_axis_name="c", subcore_axis_name="s")
    out0, out1 = pl.kernel(
        _sc_body,
        out_type=(jax.ShapeDtypeStruct((B, D), f32),
                  jax.ShapeDtypeStruct((B, D), f32)),
        mesh=mesh,
        compiler_params=pltpu.CompilerParams(
            needs_layout_passes=False, use_tc_tiling_on_sc=False),
        scratch_types=[
            pltpu.VMEM((SPB, KP), jnp.int32),
            pltpu.VMEM((SPB * KP,), f32),
            pltpu.VMEM((KP, D), jnp.bfloat16),
            pltpu.VMEM((KP, D), jnp.bfloat16),
            pltpu.VMEM((D,), f32),
            pltpu.VMEM((8, D), f32),
            pltpu.SemaphoreType.DMA,
            pltpu.SemaphoreType.DMA,
        ],
    )(fi0, fv0, fi1, fv1, w, merged_bias)
    return out0, out1


def _tp_body(in_ref, out_ref):
    out_ref[...] = in_ref[...].T.astype(jnp.bfloat16)


def _to_row_major(wT):
    return pl.pallas_call(
        _tp_body,
        grid=(pl.cdiv(NROWS, TBLK),),
        in_specs=[pl.BlockSpec((D, TBLK), lambda i: (0, i))],
        out_specs=pl.BlockSpec((TBLK, D), lambda i: (i, 0)),
        out_shape=jax.ShapeDtypeStruct((NROWS, D), jnp.bfloat16),
    )(wT)


def kernel(feature_indices_0, feature_values_0, feature_indices_1,
           feature_values_1, weight, bias_ft, bias_psqt):
    pad = ((0, 0), (0, KP - K))
    fi0 = jnp.pad(feature_indices_0, pad)
    fi1 = jnp.pad(feature_indices_1, pad)
    fv0 = jnp.pad(feature_values_0, pad).reshape(-1)
    fv1 = jnp.pad(feature_values_1, pad).reshape(-1)
    merged_bias = jnp.concatenate([bias_ft, bias_psqt], axis=0)
    w_rm = _to_row_major(weight.T)
    return _transform(fi0, fv0, fi1, fv1, w_rm, merged_bias)


# speedup vs baseline: 1.6788x; 1.2940x over previous
import jax
import jax.numpy as jnp
from jax import lax
from jax.experimental import pallas as pl
from jax.experimental.pallas import tpu as pltpu
from jax.experimental.pallas import tpu_sc as plsc

NROWS = 100000
TBLK = 2560
B = 1024
K = 50
KP = 56
D = 1032
L = 16
NC = 2
NS = 16
NW = NC * NS
SPB = B // NW
NFULL = D // L
TAIL_OFF = D - L


def _sc_body(fi0, fv0, fi1, fv1, w, bias, out0, out1,
             idxv, valv, rows0, rows1, biasv, obuf, sem0, sem1):
    wid = lax.axis_index("s") * NC + lax.axis_index("c")
    base = wid * SPB

    pltpu.sync_copy(bias, biasv)

    def run_side(fi, fv, out):
        pltpu.sync_copy(fi.at[pl.ds(base, SPB)], idxv)
        pltpu.sync_copy(fv.at[pl.ds(base * KP, SPB * KP)], valv)

        def gather(i, buf, sem):
            return pltpu.make_async_copy(w.at[idxv.at[i]], buf, sem)

        WINDOWS = [c * L for c in range(NFULL)] + [TAIL_OFF]
        HALVES = (WINDOWS[:33], WINDOWS[33:])

        def compute(rbuf, i):
            j = lax.rem(i, 8)
            vbase = jnp.full((L,), i * KP, jnp.int32)
            for half in HALVES:
                def k_body(k, accs):
                    v = plsc.load_gather(valv, [vbase + k])
                    return tuple(a + v * rbuf[k, pl.ds(o, L)]
                                 for a, o in zip(accs, half))

                init = tuple(biasv[pl.ds(o, L)] for o in half)
                accs = lax.fori_loop(0, K, k_body, init)
                for a, o in zip(accs, half):
                    obuf[j, pl.ds(o, L)] = a

            @pl.when(j == 7)
            def _():
                gb = pl.multiple_of(base + i - 7, 8)
                pltpu.sync_copy(obuf, out.at[pl.ds(gb, 8)])

        gather(0, rows0, sem0).start()

        def body(j, _):
            i = 2 * j
            gather(i + 1, rows1, sem1).start()
            gather(i, rows0, sem0).wait()
            compute(rows0, i)

            @pl.when(j < SPB // 2 - 1)
            def _():
                gather(i + 2, rows0, sem0).start()

            gather(i + 1, rows1, sem1).wait()
            compute(rows1, i + 1)
            return 0

        lax.fori_loop(0, SPB // 2, body, 0)

    run_side(fi0, fv0, out0)
    run_side(fi1, fv1, out1)


@jax.jit
def _transform(fi0, fv0, fi1, fv1, w, merged_bias):
    f32 = jnp.float32
    mesh = plsc.VectorSubcoreMesh(core_axis_name="c", subcore_axis_name="s")
    out0, out1 = pl.kernel(
        _sc_body,
        out_type=(jax.ShapeDtypeStruct((B, D), f32),
                  jax.ShapeDtypeStruct((B, D), f32)),
        mesh=mesh,
        compiler_params=pltpu.CompilerParams(
            needs_layout_passes=False, use_tc_tiling_on_sc=False),
        scratch_types=[
            pltpu.VMEM((SPB, KP), jnp.int32),
            pltpu.VMEM((SPB * KP,), f32),
            pltpu.VMEM((KP, D), f32),
            pltpu.VMEM((KP, D), f32),
            pltpu.VMEM((D,), f32),
            pltpu.VMEM((8, D), f32),
            pltpu.SemaphoreType.DMA,
            pltpu.SemaphoreType.DMA,
        ],
    )(fi0, fv0, fi1, fv1, w, merged_bias)
    return out0, out1


def _tp_body(in_ref, out_ref):
    out_ref[...] = in_ref[...].T


def _to_row_major(wT):
    return pl.pallas_call(
        _tp_body,
        grid=(pl.cdiv(NROWS, TBLK),),
        in_specs=[pl.BlockSpec((D, TBLK), lambda i: (0, i))],
        out_specs=pl.BlockSpec((TBLK, D), lambda i: (i, 0)),
        out_shape=jax.ShapeDtypeStruct((NROWS, D), jnp.float32),
    )(wT)


def kernel(feature_indices_0, feature_values_0, feature_indices_1,
           feature_values_1, weight, bias_ft, bias_psqt):
    pad = ((0, 0), (0, KP - K))
    fi0 = jnp.pad(feature_indices_0, pad)
    fi1 = jnp.pad(feature_indices_1, pad)
    fv0 = jnp.pad(feature_values_0, pad).reshape(-1)
    fv1 = jnp.pad(feature_values_1, pad).reshape(-1)
    merged_bias = jnp.concatenate([bias_ft, bias_psqt], axis=0)
    w_rm = _to_row_major(weight.T)
    return _transform(fi0, fv0, fi1, fv1, w_rm, merged_bias)
